# Initial kernel scaffold; baseline (speedup 1.0000x reference)
import jax
import jax.numpy as jnp
from jax.experimental import pallas as pl


def kernel(rpn_cls_score, rpn_bbox_pred):
    raise NotImplementedError("write your pallas kernel here")


# speedup vs baseline: 21.4457x; 21.4457x over previous
import numpy as np
import jax
import jax.numpy as jnp
from jax import lax
from jax.experimental import pallas as pl
from jax.experimental.pallas import tpu as pltpu

_FEAT_STRIDE = 16.0
_IM_H, _IM_W = 800.0, 800.0
_PRE = 6000
_POST = 300
_THRESH = 0.7
_MINSZ = 16.0
_A = 9
_H = 50
_W = 50
_N = _H * _W * _A
_ROWS = 176
_NP = _ROWS * 128


def _host_anchors() -> np.ndarray:
    scales = np.array([8.0, 16.0, 32.0])
    ratios = np.array([0.5, 1.0, 2.0])

    def mk(ws, hs, x_ctr, y_ctr):
        ws = ws[:, None]
        hs = hs[:, None]
        return np.hstack([x_ctr - 0.5 * (ws - 1), y_ctr - 0.5 * (hs - 1),
                          x_ctr + 0.5 * (ws - 1), y_ctr + 0.5 * (hs - 1)])

    base_size = 16
    base = np.array([1.0, 1.0, base_size, base_size]) - 1
    w = base[2] - base[0] + 1
    h = base[3] - base[1] + 1
    x_ctr = base[0] + 0.5 * (w - 1)
    y_ctr = base[1] + 0.5 * (h - 1)
    size = w * h
    size_ratios = size / ratios
    ws = np.round(np.sqrt(size_ratios))
    hs = np.round(ws * ratios)
    ratio_anchors = mk(ws, hs, x_ctr, y_ctr)
    out = []
    for i in range(ratio_anchors.shape[0]):
        a = ratio_anchors[i]
        aw = a[2] - a[0] + 1
        ah = a[3] - a[1] + 1
        axc = a[0] + 0.5 * (aw - 1)
        ayc = a[1] + 0.5 * (ah - 1)
        out.append(mk(aw * scales, ah * scales, axc, ayc))
    anchors_base = np.vstack(out).astype(np.float32)

    shift_x = np.arange(_W, dtype=np.float32) * _FEAT_STRIDE
    shift_y = np.arange(_H, dtype=np.float32) * _FEAT_STRIDE
    sx, sy = np.meshgrid(shift_x, shift_y)
    shifts = np.stack([sx.ravel(), sy.ravel(), sx.ravel(), sy.ravel()], axis=1)
    anchors = (shifts[:, None, :] + anchors_base[None, :, :]).reshape(-1, 4)
    return anchors.astype(np.float32)


def _pad_tile(v: np.ndarray) -> np.ndarray:
    return np.pad(v, (0, _NP - _N)).reshape(_ROWS, 128).astype(np.float32)


_ANCH = _host_anchors()
_AX1 = _pad_tile(_ANCH[:, 0])
_AY1 = _pad_tile(_ANCH[:, 1])
_AX2 = _pad_tile(_ANCH[:, 2])
_AY2 = _pad_tile(_ANCH[:, 3])


def _nms_kernel(bg, fg, dxr, dyr, dwr, dhr, ax1, ay1, ax2, ay2, out,
                x1s, y1s, x2s, y2s, ars, mss, keys):
    lin_r = lax.broadcasted_iota(jnp.int32, (_ROWS, 128), 0)
    lin_c = lax.broadcasted_iota(jnp.int32, (_ROWS, 128), 1)
    lin = lin_r * 128 + lin_c
    linf = lin.astype(jnp.float32)

    b = bg[:]
    f = fg[:]
    mx = jnp.maximum(b, f)
    eb = jnp.exp(b - mx)
    ef = jnp.exp(f - mx)
    s = ef / (eb + ef)

    widths = ax2[:] - ax1[:] + 1.0
    heights = ay2[:] - ay1[:] + 1.0
    ctr_x = ax1[:] + 0.5 * widths
    ctr_y = ay1[:] + 0.5 * heights
    pcx = dxr[:] * widths + ctr_x
    pcy = dyr[:] * heights + ctr_y
    pw = jnp.exp(dwr[:]) * widths
    ph = jnp.exp(dhr[:]) * heights
    x1 = jnp.clip(pcx - 0.5 * pw, 0.0, _IM_W - 1.0)
    y1 = jnp.clip(pcy - 0.5 * ph, 0.0, _IM_H - 1.0)
    x2 = jnp.clip(pcx + 0.5 * pw, 0.0, _IM_W - 1.0)
    y2 = jnp.clip(pcy + 0.5 * ph, 0.0, _IM_H - 1.0)
    ws = x2 - x1 + 1.0
    hs = y2 - y1 + 1.0
    s = jnp.where((ws >= _MINSZ) & (hs >= _MINSZ), s, -1e9)
    s = jnp.where(lin < _N, s, -jnp.inf)

    x1s[:] = x1
    y1s[:] = y1
    x2s[:] = x2
    y2s[:] = y2
    ars[:] = ws * hs

    bits = lax.bitcast_convert_type(s, jnp.int32)
    keys[:] = bits ^ ((bits >> 31) & jnp.int32(0x7FFFFFFF))

    def bs_val(_, lohi):
        lo, hi = lohi
        floor_avg = (lo >> 1) + (hi >> 1) + (lo & hi & 1)
        mid = floor_avg + ((lo ^ hi) & 1)
        cnt = jnp.sum(jnp.where(keys[:] >= mid, 1.0, 0.0))
        ge = cnt >= float(_PRE)
        return (jnp.where(ge, mid, lo), jnp.where(ge, hi, mid - 1))

    v_key, _ = lax.fori_loop(0, 32, bs_val,
                             (jnp.int32(-2147483648), jnp.int32(2147483647)))

    cnt_gt = jnp.sum(jnp.where(keys[:] > v_key, 1.0, 0.0))
    quota = float(_PRE) - cnt_gt

    def bs_idx(_, lohi):
        lo, hi = lohi
        mid = (lo + hi) >> 1
        cnt = jnp.sum(jnp.where((keys[:] == v_key) & (lin < mid), 1.0, 0.0))
        ge = cnt >= quota
        return (jnp.where(ge, lo, mid + 1), jnp.where(ge, mid, hi))

    i_cut, _ = lax.fori_loop(0, 15, bs_idx, (jnp.int32(0), jnp.int32(_NP)))

    elig = (keys[:] > v_key) | ((keys[:] == v_key) & (lin < i_cut))
    mss[:] = jnp.where(elig, s, -jnp.inf)

    lane = lax.broadcasted_iota(jnp.int32, (1, 128), 1)

    def nms_body(i, carry):
        ms = mss[:]
        m = jnp.max(ms)
        valid = m > -jnp.inf
        idxf = jnp.min(jnp.where(ms == m, linf, jnp.float32(3e38)))
        sel = linf == idxf
        bx1 = jnp.sum(jnp.where(sel, x1s[:], 0.0))
        by1 = jnp.sum(jnp.where(sel, y1s[:], 0.0))
        bx2 = jnp.sum(jnp.where(sel, x2s[:], 0.0))
        by2 = jnp.sum(jnp.where(sel, y2s[:], 0.0))
        bar = jnp.sum(jnp.where(sel, ars[:], 0.0))

        xx1 = jnp.maximum(x1s[:], bx1)
        yy1 = jnp.maximum(y1s[:], by1)
        xx2 = jnp.minimum(x2s[:], bx2)
        yy2 = jnp.minimum(y2s[:], by2)
        w = jnp.maximum(0.0, xx2 - xx1 + 1.0)
        h = jnp.maximum(0.0, yy2 - yy1 + 1.0)
        inter = w * h
        iou = inter / (bar + ars[:] - inter)
        mss[:] = jnp.where(iou > _THRESH, -jnp.inf, ms)

        row = (jnp.where(lane == 1, bx1, 0.0) +
               jnp.where(lane == 2, by1, 0.0) +
               jnp.where(lane == 3, bx2, 0.0) +
               jnp.where(lane == 4, by2, 0.0))
        row = row * jnp.where(valid, 1.0, 0.0)
        out[pl.ds(i, 1), :] = row
        return carry

    lax.fori_loop(0, _POST, nms_body, jnp.int32(0))


def _run(bg, fg, dxv, dyv, dwv, dhv):
    scratch = [pltpu.VMEM((_ROWS, 128), jnp.float32) for _ in range(6)]
    scratch.append(pltpu.VMEM((_ROWS, 128), jnp.int32))
    return pl.pallas_call(
        _nms_kernel,
        out_shape=jax.ShapeDtypeStruct((_POST, 128), jnp.float32),
        scratch_shapes=scratch,
    )(bg, fg, dxv, dyv, dwv, dhv,
      jnp.asarray(_AX1), jnp.asarray(_AY1), jnp.asarray(_AX2), jnp.asarray(_AY2))


def kernel(rpn_cls_score, rpn_bbox_pred):
    cls = rpn_cls_score.reshape(_H * _W, _A, 2)
    dl = rpn_bbox_pred.reshape(_H * _W, _A, 4)

    def tile(v):
        return jnp.pad(v.reshape(-1), (0, _NP - _N)).reshape(_ROWS, 128)

    bg = tile(cls[:, :, 0])
    fg = tile(cls[:, :, 1])
    dxv = tile(dl[:, :, 0])
    dyv = tile(dl[:, :, 1])
    dwv = tile(dl[:, :, 2])
    dhv = tile(dl[:, :, 3])

    out = _run(bg, fg, dxv, dyv, dwv, dhv)
    return out[:, :5]


# speedup vs baseline: 23.1443x; 1.0792x over previous
import numpy as np
import jax
import jax.numpy as jnp
from jax import lax
from jax.experimental import pallas as pl
from jax.experimental.pallas import tpu as pltpu

_FEAT_STRIDE = 16.0
_IM_H, _IM_W = 800.0, 800.0
_PRE = 6000
_POST = 300
_THRESH = 0.7
_MINSZ = 16.0
_A = 9
_H = 50
_W = 50
_N = _H * _W * _A
_ROWS = 176
_NP = _ROWS * 128


def _host_anchors() -> np.ndarray:
    scales = np.array([8.0, 16.0, 32.0])
    ratios = np.array([0.5, 1.0, 2.0])

    def mk(ws, hs, x_ctr, y_ctr):
        ws = ws[:, None]
        hs = hs[:, None]
        return np.hstack([x_ctr - 0.5 * (ws - 1), y_ctr - 0.5 * (hs - 1),
                          x_ctr + 0.5 * (ws - 1), y_ctr + 0.5 * (hs - 1)])

    base_size = 16
    base = np.array([1.0, 1.0, base_size, base_size]) - 1
    w = base[2] - base[0] + 1
    h = base[3] - base[1] + 1
    x_ctr = base[0] + 0.5 * (w - 1)
    y_ctr = base[1] + 0.5 * (h - 1)
    size = w * h
    size_ratios = size / ratios
    ws = np.round(np.sqrt(size_ratios))
    hs = np.round(ws * ratios)
    ratio_anchors = mk(ws, hs, x_ctr, y_ctr)
    out = []
    for i in range(ratio_anchors.shape[0]):
        a = ratio_anchors[i]
        aw = a[2] - a[0] + 1
        ah = a[3] - a[1] + 1
        axc = a[0] + 0.5 * (aw - 1)
        ayc = a[1] + 0.5 * (ah - 1)
        out.append(mk(aw * scales, ah * scales, axc, ayc))
    anchors_base = np.vstack(out).astype(np.float32)

    shift_x = np.arange(_W, dtype=np.float32) * _FEAT_STRIDE
    shift_y = np.arange(_H, dtype=np.float32) * _FEAT_STRIDE
    sx, sy = np.meshgrid(shift_x, shift_y)
    shifts = np.stack([sx.ravel(), sy.ravel(), sx.ravel(), sy.ravel()], axis=1)
    anchors = (shifts[:, None, :] + anchors_base[None, :, :]).reshape(-1, 4)
    return anchors.astype(np.float32)


def _pad_tile(v: np.ndarray) -> np.ndarray:
    return np.pad(v, (0, _NP - _N)).reshape(_ROWS, 128).astype(np.float32)


_ANCH = _host_anchors()
_AX1 = _pad_tile(_ANCH[:, 0])
_AY1 = _pad_tile(_ANCH[:, 1])
_AX2 = _pad_tile(_ANCH[:, 2])
_AY2 = _pad_tile(_ANCH[:, 3])


def _nms_kernel(bg, fg, dxr, dyr, dwr, dhr, ax1, ay1, ax2, ay2, out,
                x1s, y1s, x2s, y2s, ars, mss, keys):
    lin_r = lax.broadcasted_iota(jnp.int32, (_ROWS, 128), 0)
    lin_c = lax.broadcasted_iota(jnp.int32, (_ROWS, 128), 1)
    lin = lin_r * 128 + lin_c
    linf = lin.astype(jnp.float32)

    b = bg[:]
    f = fg[:]
    mx = jnp.maximum(b, f)
    eb = jnp.exp(b - mx)
    ef = jnp.exp(f - mx)
    s = ef / (eb + ef)

    widths = ax2[:] - ax1[:] + 1.0
    heights = ay2[:] - ay1[:] + 1.0
    ctr_x = ax1[:] + 0.5 * widths
    ctr_y = ay1[:] + 0.5 * heights
    pcx = dxr[:] * widths + ctr_x
    pcy = dyr[:] * heights + ctr_y
    pw = jnp.exp(dwr[:]) * widths
    ph = jnp.exp(dhr[:]) * heights
    x1 = jnp.clip(pcx - 0.5 * pw, 0.0, _IM_W - 1.0)
    y1 = jnp.clip(pcy - 0.5 * ph, 0.0, _IM_H - 1.0)
    x2 = jnp.clip(pcx + 0.5 * pw, 0.0, _IM_W - 1.0)
    y2 = jnp.clip(pcy + 0.5 * ph, 0.0, _IM_H - 1.0)
    ws = x2 - x1 + 1.0
    hs = y2 - y1 + 1.0
    s = jnp.where((ws >= _MINSZ) & (hs >= _MINSZ), s, -1e9)
    s = jnp.where(lin < _N, s, -jnp.inf)

    x1s[:] = x1
    y1s[:] = y1
    x2s[:] = x2
    y2s[:] = y2
    ars[:] = ws * hs

    bits = lax.bitcast_convert_type(s, jnp.int32)
    keys[:] = bits ^ ((bits >> 31) & jnp.int32(0x7FFFFFFF))

    def bs_val(_, lohi):
        lo, hi = lohi
        floor_avg = (lo >> 1) + (hi >> 1) + (lo & hi & 1)
        mid = floor_avg + ((lo ^ hi) & 1)
        cnt = jnp.sum(jnp.where(keys[:] >= mid, 1.0, 0.0))
        ge = cnt >= float(_PRE)
        return (jnp.where(ge, mid, lo), jnp.where(ge, hi, mid - 1))

    v_key, _ = lax.fori_loop(0, 32, bs_val,
                             (jnp.int32(-2147483648), jnp.int32(2147483647)))

    cnt_gt = jnp.sum(jnp.where(keys[:] > v_key, 1.0, 0.0))
    quota = float(_PRE) - cnt_gt

    def bs_idx(_, lohi):
        lo, hi = lohi
        mid = (lo + hi) >> 1
        cnt = jnp.sum(jnp.where((keys[:] == v_key) & (lin < mid), 1.0, 0.0))
        ge = cnt >= quota
        return (jnp.where(ge, lo, mid + 1), jnp.where(ge, mid, hi))

    i_cut, _ = lax.fori_loop(0, 15, bs_idx, (jnp.int32(0), jnp.int32(_NP)))

    elig = (keys[:] > v_key) | ((keys[:] == v_key) & (lin < i_cut))
    mss[:] = jnp.where(elig, s, -jnp.inf)

    lane = lax.broadcasted_iota(jnp.int32, (1, 128), 1)

    def nms_body(i, carry):
        ms = mss[:]
        m = jnp.max(ms)
        valid = m > -jnp.inf
        idxf = jnp.min(jnp.where(ms == m, linf, jnp.float32(3e38)))
        idx = idxf.astype(jnp.int32)
        rowi = idx >> 7
        coli = idx & 127
        lsel = lane == coli

        def pick(ref):
            return jnp.sum(jnp.where(lsel, ref[pl.ds(rowi, 1), :], 0.0))

        bx1 = pick(x1s)
        by1 = pick(y1s)
        bx2 = pick(x2s)
        by2 = pick(y2s)
        bar = pick(ars)

        xx1 = jnp.maximum(x1s[:], bx1)
        yy1 = jnp.maximum(y1s[:], by1)
        xx2 = jnp.minimum(x2s[:], bx2)
        yy2 = jnp.minimum(y2s[:], by2)
        w = jnp.maximum(0.0, xx2 - xx1 + 1.0)
        h = jnp.maximum(0.0, yy2 - yy1 + 1.0)
        inter = w * h
        iou = inter / (bar + ars[:] - inter)
        mss[:] = jnp.where(iou > _THRESH, -jnp.inf, ms)

        row = (jnp.where(lane == 1, bx1, 0.0) +
               jnp.where(lane == 2, by1, 0.0) +
               jnp.where(lane == 3, bx2, 0.0) +
               jnp.where(lane == 4, by2, 0.0))
        row = row * jnp.where(valid, 1.0, 0.0)
        out[pl.ds(i, 1), :] = row
        return carry

    lax.fori_loop(0, _POST, nms_body, jnp.int32(0))


def _run(bg, fg, dxv, dyv, dwv, dhv):
    scratch = [pltpu.VMEM((_ROWS, 128), jnp.float32) for _ in range(6)]
    scratch.append(pltpu.VMEM((_ROWS, 128), jnp.int32))
    return pl.pallas_call(
        _nms_kernel,
        out_shape=jax.ShapeDtypeStruct((_POST, 128), jnp.float32),
        scratch_shapes=scratch,
    )(bg, fg, dxv, dyv, dwv, dhv,
      jnp.asarray(_AX1), jnp.asarray(_AY1), jnp.asarray(_AX2), jnp.asarray(_AY2))


def kernel(rpn_cls_score, rpn_bbox_pred):
    cls = rpn_cls_score.reshape(_H * _W, _A, 2)
    dl = rpn_bbox_pred.reshape(_H * _W, _A, 4)

    def tile(v):
        return jnp.pad(v.reshape(-1), (0, _NP - _N)).reshape(_ROWS, 128)

    bg = tile(cls[:, :, 0])
    fg = tile(cls[:, :, 1])
    dxv = tile(dl[:, :, 0])
    dyv = tile(dl[:, :, 1])
    dwv = tile(dl[:, :, 2])
    dhv = tile(dl[:, :, 3])

    out = _run(bg, fg, dxv, dyv, dwv, dhv)
    return out[:, :5]
